# baseline (speedup 1.0000x reference)
import functools

import jax
import jax.numpy as jnp
from jax import lax
from jax.experimental import pallas as pl
from jax.experimental.pallas import tpu as pltpu
from jax.experimental.pallas import tpu_sc as plsc

_NP = 10240
_RPT = 640
_CH = 128
_NC = 2
_NS = 16
_NW = _NC * _NS
_BM = 2048


def _segsum(z, srcp, dstp):
  F = z.shape[1]
  n_chunks = srcp.shape[1]
  mesh = plsc.VectorSubcoreMesh(core_axis_name="c", subcore_axis_name="s")
  params = pltpu.CompilerParams(use_tc_tiling_on_sc=False)
  nbuf = 8
  depth = 4

  zrows = jnp.zeros((_RPT, F), jnp.float32)
  out_type = [jax.ShapeDtypeStruct((_NC, _NP, F), jnp.float32)]
  scratch = [
      pltpu.VMEM((n_chunks, _CH), jnp.int32),
      pltpu.VMEM((n_chunks, _CH), jnp.int32),
      pltpu.VMEM((nbuf, _CH, F), jnp.float32),
      pltpu.VMEM_SHARED((_NP, F), jnp.float32),
      pltpu.SemaphoreType.DMA,
      pltpu.SemaphoreType.DMA,
  ]
  def _pipeline(z_h, src_v, dst_v, rows_v, acc_sh, gsem, ssem):
    for b in range(depth):
      pltpu.async_copy(z_h.at[src_v.at[b]], rows_v.at[b], gsem)

    def chunk(j, carry):
      b = j % nbuf
      pltpu.make_async_copy(z_h.at[src_v.at[j]], rows_v.at[b], gsem).wait()
      pltpu.async_copy(rows_v.at[b], acc_sh.at[dst_v.at[j]], ssem, add=True)

      @pl.when(j >= depth)
      def _():
        pltpu.make_async_copy(rows_v.at[0], acc_sh.at[dst_v.at[0]],
                              ssem).wait()

      @pl.when(j + depth < n_chunks)
      def _():
        pltpu.async_copy(z_h.at[src_v.at[j + depth]],
                         rows_v.at[(j + depth) % nbuf], gsem)

      return carry

    lax.fori_loop(0, n_chunks, chunk, 0)
    for _ in range(depth):
      pltpu.make_async_copy(rows_v.at[0], acc_sh.at[dst_v.at[0]], ssem).wait()

  @functools.partial(pl.kernel, out_type=out_type, mesh=mesh,
                     scratch_types=scratch, compiler_params=params)
  def seg(z_h, src_h, dst_h, zr_h, acc_o,
          src_v, dst_v, rows_v, acc_sh, gsem, ssem):
    c = lax.axis_index("c")
    s = lax.axis_index("s")
    wid = s * _NC + c
    pltpu.sync_copy(zr_h, acc_sh.at[pl.ds(s * _RPT, _RPT)])
    pltpu.sync_copy(src_h.at[wid], src_v)
    pltpu.sync_copy(dst_h.at[wid], dst_v)
    plsc.subcore_barrier()
    _pipeline(z_h, src_v, dst_v, rows_v, acc_sh, gsem, ssem)
    plsc.subcore_barrier()
    sl = pl.ds(s * _RPT, _RPT)
    pltpu.sync_copy(acc_sh.at[sl], acc_o.at[c, sl])

  return seg(z, srcp, dstp, zrows)


def _deg(dstp):
  n_chunks = dstp.shape[1]
  mesh = plsc.VectorSubcoreMesh(core_axis_name="c", subcore_axis_name="s")
  params = pltpu.CompilerParams(use_tc_tiling_on_sc=False)
  zdeg = jnp.zeros((_RPT,), jnp.float32)
  ones = jnp.ones((_CH,), jnp.float32)

  @functools.partial(
      pl.kernel,
      out_type=jax.ShapeDtypeStruct((_NC, _NP), jnp.float32),
      mesh=mesh,
      scratch_types=[
          pltpu.VMEM((n_chunks, _CH), jnp.int32),
          pltpu.VMEM((_CH,), jnp.float32),
          pltpu.VMEM_SHARED((_NP,), jnp.float32),
          pltpu.SemaphoreType.DMA,
      ],
      compiler_params=params)
  def degk(dst_h, zd_h, on_h, deg_o, dst_v, ones_v, deg_sh, dsem):
    c = lax.axis_index("c")
    s = lax.axis_index("s")
    wid = s * _NC + c
    pltpu.sync_copy(zd_h, deg_sh.at[pl.ds(s * _RPT, _RPT)])
    pltpu.sync_copy(dst_h.at[wid], dst_v)
    pltpu.sync_copy(on_h, ones_v)
    plsc.subcore_barrier()

    def fire(j, carry):
      pltpu.async_copy(ones_v, deg_sh.at[dst_v.at[j]], dsem, add=True)
      return carry

    lax.fori_loop(0, n_chunks, fire, 0)

    def drain(j, carry):
      pltpu.make_async_copy(ones_v, deg_sh.at[dst_v.at[0]], dsem).wait()
      return carry

    lax.fori_loop(0, n_chunks, drain, 0)
    plsc.subcore_barrier()
    sl = pl.ds(s * _RPT, _RPT)
    pltpu.sync_copy(deg_sh.at[sl], deg_o.at[c, sl])

  return degk(dstp, zdeg, ones)


def _dotT(a, w):
  return lax.dot_general(a, w, (((1,), (1,)), ((), ())),
                         preferred_element_type=jnp.float32)


def _mm1_body(x_ref, w_ref, dd_ref, u_ref, z_ref):
  del dd_ref
  xb = x_ref[...]
  fin = xb.shape[1]
  u_ref[...] = _dotT(xb, w_ref[...][:, :fin])
  z_ref[...] = _dotT(xb, w_ref[...][:, fin:])


def _mm1(x, w1, degp):
  n, fin = x.shape
  h = w1.shape[0]
  bm = n // 5
  return pl.pallas_call(
      _mm1_body,
      grid=(5,),
      in_specs=[
          pl.BlockSpec((bm, fin), lambda i: (i, 0)),
          pl.BlockSpec((h, 2 * fin), lambda i: (0, 0)),
          pl.BlockSpec((8, _NC), lambda i: (0, 0)),
      ],
      out_specs=[
          pl.BlockSpec((bm, h), lambda i: (i, 0)),
          pl.BlockSpec((bm, h), lambda i: (i, 0)),
      ],
      out_shape=[
          jax.ShapeDtypeStruct((n, h), jnp.float32),
          jax.ShapeDtypeStruct((n, h), jnp.float32),
      ],
  )(x, w1, degp)


def _mm2_body(u_ref, a_ref, d_ref, b_ref, w_ref, u2_ref, z2_ref):
  db = d_ref[...]
  deg = db[:, 0] + db[:, 1] + 1e-6
  agg = (a_ref[0] + a_ref[1]) / deg[:, None]
  hid = jnp.maximum(u_ref[...] + agg + b_ref[...], 0.0)
  h = hid.shape[1]
  u2_ref[...] = _dotT(hid, w_ref[...][:, :h])
  z2_ref[...] = _dotT(hid, w_ref[...][:, h:])


def _mm2(u, a1, degp, b1r, w2):
  n, h = u.shape
  o = w2.shape[0]
  bm = n // 5
  return pl.pallas_call(
      _mm2_body,
      grid=(5,),
      in_specs=[
          pl.BlockSpec((bm, h), lambda i: (i, 0)),
          pl.BlockSpec((_NC, bm, h), lambda i: (0, i, 0)),
          pl.BlockSpec((bm, _NC), lambda i: (i, 0)),
          pl.BlockSpec((1, h), lambda i: (0, 0)),
          pl.BlockSpec((o, 2 * h), lambda i: (0, 0)),
      ],
      out_specs=[
          pl.BlockSpec((bm, o), lambda i: (i, 0)),
          pl.BlockSpec((bm, o), lambda i: (i, 0)),
      ],
      out_shape=[
          jax.ShapeDtypeStruct((n, o), jnp.float32),
          jax.ShapeDtypeStruct((n, o), jnp.float32),
      ],
  )(u, a1, degp, b1r, w2)


def _mm3_body(u2_ref, a_ref, d_ref, b2_ref, w3_ref, b3_ref, o_ref):
  db = d_ref[...]
  deg = db[:, 0] + db[:, 1] + 1e-6
  h2 = jnp.maximum(
      u2_ref[...] + (a_ref[0] + a_ref[1]) / deg[:, None] + b2_ref[...], 0.0)
  logit = jnp.sum(h2 * w3_ref[...], axis=1, keepdims=True) + b3_ref[...]
  o_ref[...] = jax.nn.sigmoid(logit)


def _mm3(u2, a2, degp, b2r, w3r, b3r):
  n, o = u2.shape
  bm = n // 5
  return pl.pallas_call(
      _mm3_body,
      grid=(5,),
      in_specs=[
          pl.BlockSpec((bm, o), lambda i: (i, 0)),
          pl.BlockSpec((_NC, bm, o), lambda i: (0, i, 0)),
          pl.BlockSpec((bm, _NC), lambda i: (i, 0)),
          pl.BlockSpec((1, o), lambda i: (0, 0)),
          pl.BlockSpec((1, o), lambda i: (0, 0)),
          pl.BlockSpec((1, 1), lambda i: (0, 0)),
      ],
      out_specs=pl.BlockSpec((bm, 1), lambda i: (i, 0)),
      out_shape=jax.ShapeDtypeStruct((n, 1), jnp.float32),
  )(u2, a2, degp, b2r, w3r, b3r)


def kernel(x, edge_index, W1, b1, W2, b2, W3, b3):
  n, fin = x.shape
  e = edge_index.shape[1]
  h = W1.shape[0]
  o = W2.shape[0]

  n_chunks = -(-e // (_NW * _CH))
  ep = n_chunks * _NW * _CH
  src = jnp.pad(edge_index[0], (0, ep - e)).reshape(_NW, n_chunks, _CH)
  dst = jnp.pad(edge_index[1], (0, ep - e),
                constant_values=n).reshape(_NW, n_chunks, _CH)

  d = _deg(dst)
  d = d[0] if isinstance(d, (list, tuple)) else d
  degp = d.T
  u, z = _mm1(x, W1, degp)
  a1 = _segsum(z, src, dst)
  a1 = a1[0] if isinstance(a1, (list, tuple)) else a1
  u2, z2 = _mm2(u, a1, degp, b1.reshape(1, h), W2)
  a2 = _segsum(z2, src, dst)
  a2 = a2[0] if isinstance(a2, (list, tuple)) else a2
  out = _mm3(u2, a2, degp, b2.reshape(1, o), W3, b3.reshape(1, 1))
  return out[:, 0]


# speedup vs baseline: 1.0572x; 1.0572x over previous
import functools

import jax
import jax.numpy as jnp
from jax import lax
from jax.experimental import pallas as pl
from jax.experimental.pallas import tpu as pltpu
from jax.experimental.pallas import tpu_sc as plsc

_NP = 10240
_RPT = 640
_CH = 128
_NC = 2
_NS = 16
_NW = _NC * _NS
_BM = 2048


def _segsum(z, srcp, dstp):
  F = z.shape[1]
  n_chunks = srcp.shape[1]
  mesh = plsc.VectorSubcoreMesh(core_axis_name="c", subcore_axis_name="s")
  params = pltpu.CompilerParams(use_tc_tiling_on_sc=False)
  nbuf = 8
  depth = 4

  zrows = jnp.zeros((_RPT, F), jnp.float32)
  out_type = [jax.ShapeDtypeStruct((_NC, _NP, F), jnp.float32)]
  scratch = [
      pltpu.VMEM((n_chunks, _CH), jnp.int32),
      pltpu.VMEM((n_chunks, _CH), jnp.int32),
      pltpu.VMEM((nbuf, _CH, F), jnp.float32),
      pltpu.VMEM_SHARED((_NP, F), jnp.float32),
      pltpu.SemaphoreType.DMA,
      pltpu.SemaphoreType.DMA,
  ]
  def _pipeline(z_h, src_v, dst_v, rows_v, acc_sh, gsem, ssem):
    for b in range(depth):
      pltpu.async_copy(z_h.at[src_v.at[b]], rows_v.at[b], gsem)

    def chunk(j, carry):
      b = j % nbuf
      pltpu.make_async_copy(z_h.at[src_v.at[j]], rows_v.at[b], gsem).wait()
      pltpu.async_copy(rows_v.at[b], acc_sh.at[dst_v.at[j]], ssem, add=True)

      @pl.when(j >= depth)
      def _():
        pltpu.make_async_copy(rows_v.at[0], acc_sh.at[dst_v.at[0]],
                              ssem).wait()

      @pl.when(j + depth < n_chunks)
      def _():
        pltpu.async_copy(z_h.at[src_v.at[j + depth]],
                         rows_v.at[(j + depth) % nbuf], gsem)

      return carry

    lax.fori_loop(0, n_chunks, chunk, 0)
    for _ in range(depth):
      pltpu.make_async_copy(rows_v.at[0], acc_sh.at[dst_v.at[0]], ssem).wait()

  @functools.partial(pl.kernel, out_type=out_type, mesh=mesh,
                     scratch_types=scratch, compiler_params=params)
  def seg(z_h, src_h, dst_h, zr_h, acc_o,
          src_v, dst_v, rows_v, acc_sh, gsem, ssem):
    c = lax.axis_index("c")
    s = lax.axis_index("s")
    wid = s * _NC + c
    pltpu.sync_copy(zr_h, acc_sh.at[pl.ds(s * _RPT, _RPT)])
    pltpu.sync_copy(src_h.at[wid], src_v)
    pltpu.sync_copy(dst_h.at[wid], dst_v)
    plsc.subcore_barrier()
    _pipeline(z_h, src_v, dst_v, rows_v, acc_sh, gsem, ssem)
    plsc.subcore_barrier()
    sl = pl.ds(s * _RPT, _RPT)
    pltpu.sync_copy(acc_sh.at[sl], acc_o.at[c, sl])

  return seg(z, srcp, dstp, zrows)


def _deg(dstp):
  n_chunks = dstp.shape[1]
  mesh = plsc.VectorSubcoreMesh(core_axis_name="c", subcore_axis_name="s")
  params = pltpu.CompilerParams(use_tc_tiling_on_sc=False)
  zdeg = jnp.zeros((_RPT,), jnp.float32)
  ones = jnp.ones((_CH,), jnp.float32)

  @functools.partial(
      pl.kernel,
      out_type=jax.ShapeDtypeStruct((_NC, _NP), jnp.float32),
      mesh=mesh,
      scratch_types=[
          pltpu.VMEM((n_chunks, _CH), jnp.int32),
          pltpu.VMEM((_CH,), jnp.float32),
          pltpu.VMEM_SHARED((_NP,), jnp.float32),
          pltpu.SemaphoreType.DMA,
      ],
      compiler_params=params)
  def degk(dst_h, zd_h, on_h, deg_o, dst_v, ones_v, deg_sh, dsem):
    c = lax.axis_index("c")
    s = lax.axis_index("s")
    wid = s * _NC + c
    pltpu.sync_copy(zd_h, deg_sh.at[pl.ds(s * _RPT, _RPT)])
    pltpu.sync_copy(dst_h.at[wid], dst_v)
    pltpu.sync_copy(on_h, ones_v)
    plsc.subcore_barrier()

    def fire(j, carry):
      pltpu.async_copy(ones_v, deg_sh.at[dst_v.at[j]], dsem, add=True)
      return carry

    lax.fori_loop(0, n_chunks, fire, 0)

    def drain(j, carry):
      pltpu.make_async_copy(ones_v, deg_sh.at[dst_v.at[0]], dsem).wait()
      return carry

    lax.fori_loop(0, n_chunks, drain, 0)
    plsc.subcore_barrier()
    sl = pl.ds(s * _RPT, _RPT)
    pltpu.sync_copy(deg_sh.at[sl], deg_o.at[c, sl])

  return degk(dstp, zdeg, ones)


def _dotT(a, w):
  return lax.dot_general(a, w, (((1,), (1,)), ((), ())),
                         preferred_element_type=jnp.float32)


def _mm1_body(x_ref, w_ref, u_ref, z_ref):
  xb = x_ref[...]
  fin = xb.shape[1]
  u_ref[...] = _dotT(xb, w_ref[...][:, :fin])
  z_ref[...] = _dotT(xb, w_ref[...][:, fin:])


def _mm1(x, w1):
  n, fin = x.shape
  h = w1.shape[0]
  bm = n // 5
  return pl.pallas_call(
      _mm1_body,
      grid=(5,),
      in_specs=[
          pl.BlockSpec((bm, fin), lambda i: (i, 0)),
          pl.BlockSpec((h, 2 * fin), lambda i: (0, 0)),
      ],
      out_specs=[
          pl.BlockSpec((bm, h), lambda i: (i, 0)),
          pl.BlockSpec((bm, h), lambda i: (i, 0)),
      ],
      out_shape=[
          jax.ShapeDtypeStruct((n, h), jnp.float32),
          jax.ShapeDtypeStruct((n, h), jnp.float32),
      ],
  )(x, w1)


def _mm2_body(u_ref, a_ref, d_ref, b_ref, w_ref, u2_ref, z2_ref):
  db = d_ref[...]
  deg = db[:, 0] + db[:, 1] + 1e-6
  agg = (a_ref[0] + a_ref[1]) / deg[:, None]
  hid = jnp.maximum(u_ref[...] + agg + b_ref[...], 0.0)
  h = hid.shape[1]
  u2_ref[...] = _dotT(hid, w_ref[...][:, :h])
  z2_ref[...] = _dotT(hid, w_ref[...][:, h:])


def _mm2(u, a1, degp, b1r, w2):
  n, h = u.shape
  o = w2.shape[0]
  bm = n // 5
  return pl.pallas_call(
      _mm2_body,
      grid=(5,),
      in_specs=[
          pl.BlockSpec((bm, h), lambda i: (i, 0)),
          pl.BlockSpec((_NC, bm, h), lambda i: (0, i, 0)),
          pl.BlockSpec((bm, _NC), lambda i: (i, 0)),
          pl.BlockSpec((1, h), lambda i: (0, 0)),
          pl.BlockSpec((o, 2 * h), lambda i: (0, 0)),
      ],
      out_specs=[
          pl.BlockSpec((bm, o), lambda i: (i, 0)),
          pl.BlockSpec((bm, o), lambda i: (i, 0)),
      ],
      out_shape=[
          jax.ShapeDtypeStruct((n, o), jnp.float32),
          jax.ShapeDtypeStruct((n, o), jnp.float32),
      ],
  )(u, a1, degp, b1r, w2)


def _mm3_body(u2_ref, a_ref, d_ref, b2_ref, w3_ref, b3_ref, o_ref):
  db = d_ref[...]
  deg = db[:, 0] + db[:, 1] + 1e-6
  h2 = jnp.maximum(
      u2_ref[...] + (a_ref[0] + a_ref[1]) / deg[:, None] + b2_ref[...], 0.0)
  logit = jnp.sum(h2 * w3_ref[...], axis=1, keepdims=True) + b3_ref[...]
  o_ref[...] = jax.nn.sigmoid(logit)


def _mm3(u2, a2, degp, b2r, w3r, b3r):
  n, o = u2.shape
  bm = n // 5
  return pl.pallas_call(
      _mm3_body,
      grid=(5,),
      in_specs=[
          pl.BlockSpec((bm, o), lambda i: (i, 0)),
          pl.BlockSpec((_NC, bm, o), lambda i: (0, i, 0)),
          pl.BlockSpec((bm, _NC), lambda i: (i, 0)),
          pl.BlockSpec((1, o), lambda i: (0, 0)),
          pl.BlockSpec((1, o), lambda i: (0, 0)),
          pl.BlockSpec((1, 1), lambda i: (0, 0)),
      ],
      out_specs=pl.BlockSpec((bm, 1), lambda i: (i, 0)),
      out_shape=jax.ShapeDtypeStruct((n, 1), jnp.float32),
  )(u2, a2, degp, b2r, w3r, b3r)


def kernel(x, edge_index, W1, b1, W2, b2, W3, b3):
  n, fin = x.shape
  e = edge_index.shape[1]
  h = W1.shape[0]
  o = W2.shape[0]

  n_chunks = -(-e // (_NW * _CH))
  ep = n_chunks * _NW * _CH
  src = jnp.pad(edge_index[0], (0, ep - e)).reshape(_NW, n_chunks, _CH)
  dst = jnp.pad(edge_index[1], (0, ep - e),
                constant_values=n).reshape(_NW, n_chunks, _CH)

  d = _deg(dst)
  d = d[0] if isinstance(d, (list, tuple)) else d
  degp = d.T
  u, z = _mm1(x, W1)
  a1 = _segsum(z, src, dst)
  a1 = a1[0] if isinstance(a1, (list, tuple)) else a1
  u2, z2 = _mm2(u, a1, degp, b1.reshape(1, h), W2)
  a2 = _segsum(z2, src, dst)
  a2 = a2[0] if isinstance(a2, (list, tuple)) else a2
  out = _mm3(u2, a2, degp, b2.reshape(1, o), W3, b3.reshape(1, 1))
  return out[:, 0]


# speedup vs baseline: 1.1049x; 1.0451x over previous
import functools

import jax
import jax.numpy as jnp
from jax import lax
from jax.experimental import pallas as pl
from jax.experimental.pallas import tpu as pltpu
from jax.experimental.pallas import tpu_sc as plsc

_NP = 10240
_RPT = 640
_CH = 128
_NC = 2
_NS = 16
_NW = _NC * _NS
_BM = 2048


def _segsum(z, srcp, dstp):
  F = z.shape[1]
  n_chunks = srcp.shape[1]
  mesh = plsc.VectorSubcoreMesh(core_axis_name="c", subcore_axis_name="s")
  params = pltpu.CompilerParams(use_tc_tiling_on_sc=False)
  nbuf = 8
  depth = 4

  zrows = jnp.zeros((_RPT, F), jnp.float32)
  out_type = [jax.ShapeDtypeStruct((_NC, _NP, F), jnp.float32)]
  scratch = [
      pltpu.VMEM((n_chunks, _CH), jnp.int32),
      pltpu.VMEM((n_chunks, _CH), jnp.int32),
      pltpu.VMEM((nbuf, _CH, F), jnp.float32),
      pltpu.VMEM_SHARED((_NP, F), jnp.float32),
      pltpu.SemaphoreType.DMA,
      pltpu.SemaphoreType.DMA,
  ]
  def _pipeline(z_h, src_v, dst_v, rows_v, acc_sh, gsem, ssem):
    for b in range(depth):
      pltpu.async_copy(z_h.at[src_v.at[b]], rows_v.at[b], gsem)

    def chunk(j, carry):
      b = j % nbuf
      pltpu.make_async_copy(z_h.at[src_v.at[j]], rows_v.at[b], gsem).wait()
      pltpu.async_copy(rows_v.at[b], acc_sh.at[dst_v.at[j]], ssem, add=True)

      @pl.when(j >= depth)
      def _():
        pltpu.make_async_copy(rows_v.at[0], acc_sh.at[dst_v.at[0]],
                              ssem).wait()

      @pl.when(j + depth < n_chunks)
      def _():
        pltpu.async_copy(z_h.at[src_v.at[j + depth]],
                         rows_v.at[(j + depth) % nbuf], gsem)

      return carry

    lax.fori_loop(0, n_chunks, chunk, 0)
    for _ in range(depth):
      pltpu.make_async_copy(rows_v.at[0], acc_sh.at[dst_v.at[0]], ssem).wait()

  @functools.partial(pl.kernel, out_type=out_type, mesh=mesh,
                     scratch_types=scratch, compiler_params=params)
  def seg(z_h, src_h, dst_h, zr_h, acc_o,
          src_v, dst_v, rows_v, acc_sh, gsem, ssem):
    c = lax.axis_index("c")
    s = lax.axis_index("s")
    wid = s * _NC + c
    pltpu.sync_copy(zr_h, acc_sh.at[pl.ds(s * _RPT, _RPT)])
    pltpu.sync_copy(src_h.at[wid], src_v)
    pltpu.sync_copy(dst_h.at[wid], dst_v)
    plsc.subcore_barrier()
    _pipeline(z_h, src_v, dst_v, rows_v, acc_sh, gsem, ssem)
    plsc.subcore_barrier()
    sl = pl.ds(s * _RPT, _RPT)
    pltpu.sync_copy(acc_sh.at[sl], acc_o.at[c, sl])

  return seg(z, srcp, dstp, zrows)


def _deg(dstp):
  n_chunks = dstp.shape[1]
  mesh = plsc.VectorSubcoreMesh(core_axis_name="c", subcore_axis_name="s")
  params = pltpu.CompilerParams(use_tc_tiling_on_sc=False)
  zdeg = jnp.zeros((_RPT,), jnp.float32)
  ones = jnp.ones((_CH,), jnp.float32)

  @functools.partial(
      pl.kernel,
      out_type=jax.ShapeDtypeStruct((_NC, _NP), jnp.float32),
      mesh=mesh,
      scratch_types=[
          pltpu.VMEM((n_chunks, _CH), jnp.int32),
          pltpu.VMEM((_CH,), jnp.float32),
          pltpu.VMEM_SHARED((_NP,), jnp.float32),
          pltpu.SemaphoreType.DMA,
      ],
      compiler_params=params)
  def degk(dst_h, zd_h, on_h, deg_o, dst_v, ones_v, deg_sh, dsem):
    c = lax.axis_index("c")
    s = lax.axis_index("s")
    wid = s * _NC + c
    pltpu.sync_copy(zd_h, deg_sh.at[pl.ds(s * _RPT, _RPT)])
    pltpu.sync_copy(dst_h.at[wid], dst_v)
    pltpu.sync_copy(on_h, ones_v)
    plsc.subcore_barrier()

    def fire(j, carry):
      pltpu.async_copy(ones_v, deg_sh.at[dst_v.at[j]], dsem, add=True)
      return carry

    lax.fori_loop(0, n_chunks, fire, 0)

    def drain(j, carry):
      pltpu.make_async_copy(ones_v, deg_sh.at[dst_v.at[0]], dsem).wait()
      return carry

    lax.fori_loop(0, n_chunks, drain, 0)
    plsc.subcore_barrier()
    sl = pl.ds(s * _RPT, _RPT)
    pltpu.sync_copy(deg_sh.at[sl], deg_o.at[c, sl])

  return degk(dstp, zdeg, ones)


def _mm1_body(x_ref, wa_ref, wb_ref, u_ref, z_ref):
  xb = x_ref[...]
  u_ref[...] = jnp.dot(xb, wa_ref[...], preferred_element_type=jnp.float32)
  z_ref[...] = jnp.dot(xb, wb_ref[...], preferred_element_type=jnp.float32)


def _mm1(x, wa, wb):
  n, fin = x.shape
  h = wa.shape[1]
  bm = n // 5
  return pl.pallas_call(
      _mm1_body,
      grid=(5,),
      in_specs=[
          pl.BlockSpec((bm, fin), lambda i: (i, 0)),
          pl.BlockSpec((fin, h), lambda i: (0, 0)),
          pl.BlockSpec((fin, h), lambda i: (0, 0)),
      ],
      out_specs=[
          pl.BlockSpec((bm, h), lambda i: (i, 0)),
          pl.BlockSpec((bm, h), lambda i: (i, 0)),
      ],
      out_shape=[
          jax.ShapeDtypeStruct((n, h), jnp.float32),
          jax.ShapeDtypeStruct((n, h), jnp.float32),
      ],
  )(x, wa, wb)


def _mm2_body(u_ref, a_ref, d_ref, b_ref, w_ref, u2_ref, z2_ref):
  db = d_ref[...]
  deg = db[:, 0] + db[:, 1] + 1e-6
  agg = (a_ref[0] + a_ref[1]) / deg[:, None]
  hid = jnp.maximum(u_ref[...] + agg + b_ref[...], 0.0)
  hz = jnp.dot(hid, w_ref[...], preferred_element_type=jnp.float32)
  o = hz.shape[1] // 2
  u2_ref[...] = hz[:, :o]
  z2_ref[...] = hz[:, o:]


def _mm2(u, a1, degp, b1r, wc2):
  n, h = u.shape
  o = wc2.shape[1] // 2
  bm = n // 5
  return pl.pallas_call(
      _mm2_body,
      grid=(5,),
      in_specs=[
          pl.BlockSpec((bm, h), lambda i: (i, 0)),
          pl.BlockSpec((_NC, bm, h), lambda i: (0, i, 0)),
          pl.BlockSpec((bm, _NC), lambda i: (i, 0)),
          pl.BlockSpec((1, h), lambda i: (0, 0)),
          pl.BlockSpec((h, 2 * o), lambda i: (0, 0)),
      ],
      out_specs=[
          pl.BlockSpec((bm, o), lambda i: (i, 0)),
          pl.BlockSpec((bm, o), lambda i: (i, 0)),
      ],
      out_shape=[
          jax.ShapeDtypeStruct((n, o), jnp.float32),
          jax.ShapeDtypeStruct((n, o), jnp.float32),
      ],
  )(u, a1, degp, b1r, wc2)


def _mm3_body(u2_ref, a_ref, d_ref, b2_ref, w3_ref, b3_ref, o_ref):
  db = d_ref[...]
  deg = db[:, 0] + db[:, 1] + 1e-6
  h2 = jnp.maximum(
      u2_ref[...] + (a_ref[0] + a_ref[1]) / deg[:, None] + b2_ref[...], 0.0)
  logit = jnp.sum(h2 * w3_ref[...], axis=1, keepdims=True) + b3_ref[...]
  o_ref[...] = jax.nn.sigmoid(logit)


def _mm3(u2, a2, degp, b2r, w3r, b3r):
  n, o = u2.shape
  bm = n // 5
  return pl.pallas_call(
      _mm3_body,
      grid=(5,),
      in_specs=[
          pl.BlockSpec((bm, o), lambda i: (i, 0)),
          pl.BlockSpec((_NC, bm, o), lambda i: (0, i, 0)),
          pl.BlockSpec((bm, _NC), lambda i: (i, 0)),
          pl.BlockSpec((1, o), lambda i: (0, 0)),
          pl.BlockSpec((1, o), lambda i: (0, 0)),
          pl.BlockSpec((1, 1), lambda i: (0, 0)),
      ],
      out_specs=pl.BlockSpec((bm, 1), lambda i: (i, 0)),
      out_shape=jax.ShapeDtypeStruct((n, 1), jnp.float32),
  )(u2, a2, degp, b2r, w3r, b3r)


def kernel(x, edge_index, W1, b1, W2, b2, W3, b3):
  n, fin = x.shape
  e = edge_index.shape[1]
  h = W1.shape[0]
  o = W2.shape[0]

  n_chunks = -(-e // (_NW * _CH))
  ep = n_chunks * _NW * _CH
  src = jnp.pad(edge_index[0], (0, ep - e)).reshape(_NW, n_chunks, _CH)
  dst = jnp.pad(edge_index[1], (0, ep - e),
                constant_values=n).reshape(_NW, n_chunks, _CH)

  wa1 = W1[:, :fin].T
  wb1 = W1[:, fin:].T
  wc2 = jnp.concatenate([W2[:, :h].T, W2[:, h:].T], axis=1)

  d = _deg(dst)
  d = d[0] if isinstance(d, (list, tuple)) else d
  degp = d.T
  u, z = _mm1(x, wa1, wb1)
  a1 = _segsum(z, src, dst)
  a1 = a1[0] if isinstance(a1, (list, tuple)) else a1
  u2, z2 = _mm2(u, a1, degp, b1.reshape(1, h), wc2)
  a2 = _segsum(z2, src, dst)
  a2 = a2[0] if isinstance(a2, (list, tuple)) else a2
  out = _mm3(u2, a2, degp, b2.reshape(1, o), W3, b3.reshape(1, 1))
  return out[:, 0]


# speedup vs baseline: 1.1488x; 1.0397x over previous
import functools

import jax
import jax.numpy as jnp
from jax import lax
from jax.experimental import pallas as pl
from jax.experimental.pallas import tpu as pltpu
from jax.experimental.pallas import tpu_sc as plsc

_NP = 10240
_RPT = 640
_CH = 128
_NC = 2
_NS = 16
_NW = _NC * _NS
_BM = 2048


def _segsum(z, srcp, dstp):
  F = z.shape[1]
  n_chunks = srcp.shape[1]
  mesh = plsc.VectorSubcoreMesh(core_axis_name="c", subcore_axis_name="s")
  params = pltpu.CompilerParams(use_tc_tiling_on_sc=False)
  nbuf = 8
  depth = 4

  zrows = jnp.zeros((_RPT, F), jnp.float32)
  out_type = [jax.ShapeDtypeStruct((_NC, _NP, F), jnp.float32)]
  scratch = [
      pltpu.VMEM((n_chunks, _CH), jnp.int32),
      pltpu.VMEM((n_chunks, _CH), jnp.int32),
      pltpu.VMEM((nbuf, _CH, F), jnp.float32),
      pltpu.VMEM_SHARED((_NP, F), jnp.float32),
      pltpu.SemaphoreType.DMA,
      pltpu.SemaphoreType.DMA,
  ]
  def _pipeline(z_h, src_v, dst_v, rows_v, acc_sh, gsem, ssem):
    for b in range(depth):
      pltpu.async_copy(z_h.at[src_v.at[b]], rows_v.at[b], gsem)

    def chunk(j, carry):
      b = j % nbuf
      pltpu.make_async_copy(z_h.at[src_v.at[j]], rows_v.at[b], gsem).wait()
      pltpu.async_copy(rows_v.at[b], acc_sh.at[dst_v.at[j]], ssem, add=True)

      @pl.when(j >= depth)
      def _():
        pltpu.make_async_copy(rows_v.at[0], acc_sh.at[dst_v.at[0]],
                              ssem).wait()

      @pl.when(j + depth < n_chunks)
      def _():
        pltpu.async_copy(z_h.at[src_v.at[j + depth]],
                         rows_v.at[(j + depth) % nbuf], gsem)

      return carry

    lax.fori_loop(0, n_chunks, chunk, 0)
    for _ in range(depth):
      pltpu.make_async_copy(rows_v.at[0], acc_sh.at[dst_v.at[0]], ssem).wait()

  @functools.partial(pl.kernel, out_type=out_type, mesh=mesh,
                     scratch_types=scratch, compiler_params=params)
  def seg(z_h, src_h, dst_h, zr_h, acc_o,
          src_v, dst_v, rows_v, acc_sh, gsem, ssem):
    c = lax.axis_index("c")
    s = lax.axis_index("s")
    wid = s * _NC + c
    pltpu.sync_copy(zr_h, acc_sh.at[pl.ds(s * _RPT, _RPT)])
    pltpu.sync_copy(src_h.at[wid], src_v)
    pltpu.sync_copy(dst_h.at[wid], dst_v)
    plsc.subcore_barrier()
    _pipeline(z_h, src_v, dst_v, rows_v, acc_sh, gsem, ssem)
    plsc.subcore_barrier()
    sl = pl.ds(s * _RPT, _RPT)
    pltpu.sync_copy(acc_sh.at[sl], acc_o.at[c, sl])

  return seg(z, srcp, dstp, zrows)


def _deg(dstp):
  n_chunks = dstp.shape[1]
  mesh = plsc.VectorSubcoreMesh(core_axis_name="c", subcore_axis_name="s")
  params = pltpu.CompilerParams(use_tc_tiling_on_sc=False)
  zdeg = jnp.zeros((_RPT,), jnp.float32)
  ones = jnp.ones((_CH,), jnp.float32)

  @functools.partial(
      pl.kernel,
      out_type=jax.ShapeDtypeStruct((_NC, _NP), jnp.float32),
      mesh=mesh,
      scratch_types=[
          pltpu.VMEM((n_chunks, _CH), jnp.int32),
          pltpu.VMEM((_CH,), jnp.float32),
          pltpu.VMEM_SHARED((_NP,), jnp.float32),
          pltpu.SemaphoreType.DMA,
      ],
      compiler_params=params)
  def degk(dst_h, zd_h, on_h, deg_o, dst_v, ones_v, deg_sh, dsem):
    c = lax.axis_index("c")
    s = lax.axis_index("s")
    wid = s * _NC + c
    pltpu.sync_copy(zd_h, deg_sh.at[pl.ds(s * _RPT, _RPT)])
    pltpu.sync_copy(dst_h.at[wid], dst_v)
    pltpu.sync_copy(on_h, ones_v)
    plsc.subcore_barrier()

    def fire(j, carry):
      pltpu.async_copy(ones_v, deg_sh.at[dst_v.at[j]], dsem, add=True)
      return carry

    lax.fori_loop(0, n_chunks, fire, 0)

    def drain(j, carry):
      pltpu.make_async_copy(ones_v, deg_sh.at[dst_v.at[0]], dsem).wait()
      return carry

    lax.fori_loop(0, n_chunks, drain, 0)
    plsc.subcore_barrier()
    sl = pl.ds(s * _RPT, _RPT)
    pltpu.sync_copy(deg_sh.at[sl], deg_o.at[c, sl])

  return degk(dstp, zdeg, ones)


def _mm1_body(x_ref, wa_ref, wb_ref, u_ref, z_ref):
  xb = x_ref[...]
  u_ref[...] = jnp.dot(xb, wa_ref[...], preferred_element_type=jnp.float32)
  z_ref[...] = jnp.dot(xb, wb_ref[...], preferred_element_type=jnp.float32)


def _mm1(x, wa, wb):
  n, fin = x.shape
  h = wa.shape[1]
  bm = n // 5
  return pl.pallas_call(
      _mm1_body,
      grid=(5,),
      in_specs=[
          pl.BlockSpec((bm, fin), lambda i: (i, 0)),
          pl.BlockSpec((fin, h), lambda i: (0, 0)),
          pl.BlockSpec((fin, h), lambda i: (0, 0)),
      ],
      out_specs=[
          pl.BlockSpec((bm, h), lambda i: (i, 0)),
          pl.BlockSpec((bm, h), lambda i: (i, 0)),
      ],
      out_shape=[
          jax.ShapeDtypeStruct((n, h), jnp.float32),
          jax.ShapeDtypeStruct((n, h), jnp.float32),
      ],
  )(x, wa, wb)


def _mm2_body(u_ref, a_ref, d_ref, b_ref, w_ref, u2_ref, z2_ref):
  db = d_ref[...]
  deg = db[:, 0] + db[:, 1] + 1e-6
  agg = (a_ref[0] + a_ref[1]) / deg[:, None]
  hid = jnp.maximum(u_ref[...] + agg + b_ref[...], 0.0)
  hz = jnp.dot(hid, w_ref[...], preferred_element_type=jnp.float32)
  o = hz.shape[1] // 2
  u2_ref[...] = hz[:, :o]
  z2_ref[...] = hz[:, o:]


def _mm2(u, a1, degp, b1r, wc2):
  n, h = u.shape
  o = wc2.shape[1] // 2
  bm = n // 5
  return pl.pallas_call(
      _mm2_body,
      grid=(5,),
      in_specs=[
          pl.BlockSpec((bm, h), lambda i: (i, 0)),
          pl.BlockSpec((_NC, bm, h), lambda i: (0, i, 0)),
          pl.BlockSpec((bm, _NC), lambda i: (i, 0)),
          pl.BlockSpec((1, h), lambda i: (0, 0)),
          pl.BlockSpec((h, 2 * o), lambda i: (0, 0)),
      ],
      out_specs=[
          pl.BlockSpec((bm, o), lambda i: (i, 0)),
          pl.BlockSpec((bm, o), lambda i: (i, 0)),
      ],
      out_shape=[
          jax.ShapeDtypeStruct((n, o), jnp.float32),
          jax.ShapeDtypeStruct((n, o), jnp.float32),
      ],
  )(u, a1, degp, b1r, wc2)


def _mm3_body(u2_ref, a_ref, d_ref, b2_ref, w3_ref, b3_ref, o_ref):
  db = d_ref[...]
  deg = db[:, 0] + db[:, 1] + 1e-6
  h2 = jnp.maximum(
      u2_ref[...] + (a_ref[0] + a_ref[1]) / deg[:, None] + b2_ref[...], 0.0)
  logit = jnp.sum(h2 * w3_ref[...], axis=1, keepdims=True) + b3_ref[...]
  o_ref[...] = jax.nn.sigmoid(logit)


def _mm3(u2, a2, degp, b2r, w3r, b3r):
  n, o = u2.shape
  bm = n // 5
  return pl.pallas_call(
      _mm3_body,
      grid=(5,),
      in_specs=[
          pl.BlockSpec((bm, o), lambda i: (i, 0)),
          pl.BlockSpec((_NC, bm, o), lambda i: (0, i, 0)),
          pl.BlockSpec((bm, _NC), lambda i: (i, 0)),
          pl.BlockSpec((1, o), lambda i: (0, 0)),
          pl.BlockSpec((1, o), lambda i: (0, 0)),
          pl.BlockSpec((1, 1), lambda i: (0, 0)),
      ],
      out_specs=pl.BlockSpec((bm, 1), lambda i: (i, 0)),
      out_shape=jax.ShapeDtypeStruct((n, 1), jnp.float32),
  )(u2, a2, degp, b2r, w3r, b3r)


def kernel(x, edge_index, W1, b1, W2, b2, W3, b3):
  n, fin = x.shape
  e = edge_index.shape[1]
  h = W1.shape[0]
  o = W2.shape[0]

  n_chunks = -(-e // (_NW * _CH))
  ep = n_chunks * _NW * _CH
  src = jnp.pad(edge_index[0], (0, ep - e)).reshape(_NW, n_chunks, _CH)
  dst = jnp.pad(edge_index[1], (0, ep - e),
                constant_values=n).reshape(_NW, n_chunks, _CH)

  wa1 = W1[:, :fin].T
  wb1 = W1[:, fin:].T
  wc2 = jnp.concatenate([W2[:, :h].T, W2[:, h:].T], axis=1)

  xp = jnp.pad(x, ((0, _NP - n), (0, 0)))
  d = _deg(dst)
  d = d[0] if isinstance(d, (list, tuple)) else d
  degp = d.T
  u, z = _mm1(xp, wa1, wb1)
  a1 = _segsum(z, src, dst)
  a1 = a1[0] if isinstance(a1, (list, tuple)) else a1
  u2, z2 = _mm2(u, a1, degp, b1.reshape(1, h), wc2)
  a2 = _segsum(z2, src, dst)
  a2 = a2[0] if isinstance(a2, (list, tuple)) else a2
  out = _mm3(u2, a2, degp, b2.reshape(1, o), W3, b3.reshape(1, 1))
  return out[:n, 0]
